# baseline (speedup 1.0000x reference)
import jax
import jax.numpy as jnp
from jax import lax
from jax.experimental import pallas as pl
from jax.experimental.pallas import tpu as pltpu
from jax.experimental.pallas import tpu_sc as plsc

_N = 10000
_E = 320000
_D = 128
_H = 128
_HW = _H // 2

_NC = 2
_NS = 16
_NW = _NC * _NS

_SLICES = ((320000, 80, 125, 8000),)
_LN_EPS = 1e-5


def _pack2(x_f32):
    lo = lax.bitcast_convert_type(
        x_f32[:, :_HW].astype(jnp.bfloat16), jnp.uint16).astype(jnp.uint32)
    hi = lax.bitcast_convert_type(
        x_f32[:, _HW:].astype(jnp.bfloat16), jnp.uint16).astype(jnp.uint32)
    return lax.bitcast_convert_type(lo | (hi << 16), jnp.int32)


def _unpack2(w_i32):
    w_u32 = lax.bitcast_convert_type(w_i32, jnp.uint32)
    lo = lax.bitcast_convert_type(
        (w_u32 & jnp.uint32(0xFFFF)).astype(jnp.uint16), jnp.bfloat16)
    hi = lax.bitcast_convert_type(
        (w_u32 >> 16).astype(jnp.uint16), jnp.bfloat16)
    return jnp.concatenate(
        [lo.astype(jnp.float32), hi.astype(jnp.float32)], axis=-1)


def _proj_body(nf_ref, wst_ref, wdt_ref, s_ref, d_ref):
    nf = nf_ref[...]
    s_ref[...] = _pack2(
        jnp.dot(nf, wst_ref[...], preferred_element_type=jnp.float32))
    d_ref[...] = _pack2(
        jnp.dot(nf, wdt_ref[...], preferred_element_type=jnp.float32))


def _proj_call(nfeat, wst, wdt):
    return pl.pallas_call(
        _proj_body,
        out_shape=(
            jax.ShapeDtypeStruct((_N, _HW), jnp.int32),
            jax.ShapeDtypeStruct((_N, _HW), jnp.int32),
        ),
    )(nfeat, wst, wdt)


def _gather_call(mlp_s, mlp_d, src, dst, e_slice, c, n_chunks):
    per_w = e_slice // _NW
    assert c % 8 == 0 and 0 < c <= 128 and per_w == c * n_chunks
    assert n_chunks % 2 == 1 and n_chunks >= 5

    def body(s_hbm, d_hbm, src_hbm, dst_hbm, out_hbm,
             idx_s, idx_d, buf_s, buf_d, obuf, sem_s, sem_d, sem_o):
        wid = lax.axis_index("s") * _NC + lax.axis_index("c")
        base = wid * per_w

        pltpu.sync_copy(src_hbm.at[pl.ds(pl.multiple_of(base, 8), per_w)],
                        idx_s)
        pltpu.sync_copy(dst_hbm.at[pl.ds(pl.multiple_of(base, 8), per_w)],
                        idx_d)

        def issue(j, slot):
            js = pl.multiple_of(j * c, 8)
            pltpu.async_copy(s_hbm.at[idx_s.at[pl.ds(js, c)]], buf_s.at[slot],
                             sem_s.at[slot])
            pltpu.async_copy(d_hbm.at[idx_d.at[pl.ds(js, c)]], buf_d.at[slot],
                             sem_d.at[slot])

        def process(j, slot, first):
            off = pl.multiple_of(base + j * c, 8)
            pltpu.make_async_copy(s_hbm.at[idx_s.at[pl.ds(0, c)]],
                                  buf_s.at[slot], sem_s.at[slot]).wait()
            pltpu.make_async_copy(d_hbm.at[idx_d.at[pl.ds(0, c)]],
                                  buf_d.at[slot], sem_d.at[slot]).wait()
            if not first:
                pltpu.make_async_copy(obuf.at[slot],
                                      out_hbm.at[pl.ds(0, c // 2)],
                                      sem_o.at[slot]).wait()

            def row(p, c2):
                for q in range(2):
                    e = p * 2 + q
                    for k in range(_HW // 16):
                        a = plsc.bitcast(buf_s[slot, e, pl.ds(k * 16, 16)],
                                         jnp.bfloat16)
                        b = plsc.bitcast(buf_d[slot, e, pl.ds(k * 16, 16)],
                                         jnp.bfloat16)
                        obuf[slot, p, pl.ds(q * _HW + k * 16, 16)] = (
                            plsc.bitcast(a + b, jnp.int32))
                return c2

            lax.fori_loop(0, c // 2, row, 0)
            pltpu.async_copy(obuf.at[slot],
                             out_hbm.at[pl.ds(off // 2, c // 2)],
                             sem_o.at[slot])

        issue(0, 0)
        issue(1, 1)
        process(0, 0, first=True)
        issue(2, 0)
        process(1, 1, first=True)
        issue(3, 1)

        def pair(k, carry):
            j = k * 2
            process(j, 0, first=False)
            issue(j + 2, 0)
            process(j + 1, 1, first=False)
            issue(j + 3, 1)
            return carry

        lax.fori_loop(1, (n_chunks - 3) // 2, pair, 0)
        process(n_chunks - 3, 0, first=False)
        issue(n_chunks - 1, 0)
        process(n_chunks - 2, 1, first=False)
        process(n_chunks - 1, 0, first=False)
        pltpu.make_async_copy(obuf.at[0], out_hbm.at[pl.ds(0, c // 2)],
                              sem_o.at[0]).wait()
        pltpu.make_async_copy(obuf.at[1], out_hbm.at[pl.ds(0, c // 2)],
                              sem_o.at[1]).wait()

    mesh = plsc.VectorSubcoreMesh(
        core_axis_name="c", subcore_axis_name="s",
        num_cores=_NC, num_subcores=_NS)
    fn = pl.kernel(
        body,
        out_type=jax.ShapeDtypeStruct((e_slice // 2, _H), jnp.int32),
        mesh=mesh,
        compiler_params=pltpu.CompilerParams(
            use_tc_tiling_on_sc=False, needs_layout_passes=False),
        scratch_types=[
            pltpu.VMEM((per_w,), jnp.int32),
            pltpu.VMEM((per_w,), jnp.int32),
            pltpu.VMEM((2, c, _HW), jnp.int32),
            pltpu.VMEM((2, c, _HW), jnp.int32),
            pltpu.VMEM((2, c // 2, _H), jnp.int32),
            pltpu.SemaphoreType.DMA((2,)),
            pltpu.SemaphoreType.DMA((2,)),
            pltpu.SemaphoreType.DMA((2,)),
        ],
    )
    return fn(mlp_s, mlp_d, src, dst)


def _edge_math(e, gg, wet, wot, b1, bo, lng, lnb):
    even = _unpack2(gg[:, :_HW])
    odd = _unpack2(gg[:, _HW:])
    gnat = jnp.concatenate([even, odd], axis=-1).reshape(2 * gg.shape[0], _D)
    h = jnp.dot(e, wet, preferred_element_type=jnp.float32)
    h = h + gnat + b1
    h = h / (1.0 + jnp.exp(-h))
    o = jnp.dot(h, wot, preferred_element_type=jnp.float32)
    o = o + bo
    m = jnp.mean(o, axis=-1, keepdims=True)
    v = jnp.mean(o * o, axis=-1, keepdims=True) - m * m
    o = (o - m) * lax.rsqrt(v + _LN_EPS) * lng + lnb
    return o + e


def _edge_body_first(e_ref, g_ref, wet_ref, wot_ref, b1_ref, bo_ref,
                     lng_ref, lnb_ref, out_ref):
    out_ref[...] = _edge_math(
        e_ref[...], g_ref[...], wet_ref[...], wot_ref[...], b1_ref[...],
        bo_ref[...], lng_ref[...], lnb_ref[...])


def _edge_body_rest(prev_ref, e_ref, g_ref, wet_ref, wot_ref, b1_ref, bo_ref,
                    lng_ref, lnb_ref, out_ref):
    del prev_ref
    out_ref[...] = _edge_math(
        e_ref[...], g_ref[...], wet_ref[...], wot_ref[...], b1_ref[...],
        bo_ref[...], lng_ref[...], lnb_ref[...])


def _edge_call(efeat, g, wk, prev_out, e_off, be, n_blks):
    blk_off = e_off // be
    blk = pl.BlockSpec((be, _D), lambda i: (i + blk_off, 0))
    gblk = pl.BlockSpec((be // 2, _H), lambda i: (i, 0))
    full = pl.BlockSpec((_D, _H), lambda i: (0, 0))
    vec = pl.BlockSpec((1, _H), lambda i: (0, 0))
    out_shape = jax.ShapeDtypeStruct((_E, _D), jnp.float32)
    if prev_out is None:
        return pl.pallas_call(
            _edge_body_first,
            grid=(n_blks,),
            in_specs=[blk, gblk, full, full, vec, vec, vec, vec],
            out_specs=blk,
            out_shape=out_shape,
        )(efeat, g, *wk)
    return pl.pallas_call(
        _edge_body_rest,
        grid=(n_blks,),
        in_specs=[pl.BlockSpec(memory_space=pl.ANY),
                  blk, gblk, full, full, vec, vec, vec, vec],
        out_specs=blk,
        out_shape=out_shape,
        input_output_aliases={0: 0},
    )(prev_out, efeat, g, *wk)


def kernel(efeat, nfeat, src, dst, W_e, W_s, W_d, b1, W_o, b_o, ln_g, ln_b):
    mlp_s, mlp_d = _proj_call(nfeat, W_s.T, W_d.T)
    wk = (W_e.T, W_o.T, b1.reshape(1, _H), b_o.reshape(1, _D),
          ln_g.reshape(1, _D), ln_b.reshape(1, _D))
    gs = []
    e0 = 0
    for es, c, nch, _ in _SLICES:
        gs.append(_gather_call(mlp_s, mlp_d, src[e0:e0 + es],
                               dst[e0:e0 + es], es, c, nch))
        e0 += es
    out = None
    e0 = 0
    for k, (es, _, _, be) in enumerate(_SLICES):
        out = _edge_call(efeat, gs[k], wk, out, e0, be, es // be)
        e0 += es
    return (out, nfeat)


# speedup vs baseline: 1.0312x; 1.0312x over previous
import jax
import jax.numpy as jnp
from jax import lax
from jax.experimental import pallas as pl
from jax.experimental.pallas import tpu as pltpu
from jax.experimental.pallas import tpu_sc as plsc

_N = 10000
_E = 320000
_D = 128
_H = 128
_HW = _H // 2

_NC = 2
_NS = 16
_NW = _NC * _NS

_SLICES = ((320000, 128, 79, 8000, 16),)
_LN_EPS = 1e-5


def _pack2(x_f32):
    lo = lax.bitcast_convert_type(
        x_f32[:, :_HW].astype(jnp.bfloat16), jnp.uint16).astype(jnp.uint32)
    hi = lax.bitcast_convert_type(
        x_f32[:, _HW:].astype(jnp.bfloat16), jnp.uint16).astype(jnp.uint32)
    return lax.bitcast_convert_type(lo | (hi << 16), jnp.int32)


def _unpack2(w_i32):
    w_u32 = lax.bitcast_convert_type(w_i32, jnp.uint32)
    lo = lax.bitcast_convert_type(
        (w_u32 & jnp.uint32(0xFFFF)).astype(jnp.uint16), jnp.bfloat16)
    hi = lax.bitcast_convert_type(
        (w_u32 >> 16).astype(jnp.uint16), jnp.bfloat16)
    return jnp.concatenate(
        [lo.astype(jnp.float32), hi.astype(jnp.float32)], axis=-1)


def _proj_body(nf_ref, wst_ref, wdt_ref, s_ref, d_ref):
    nf = nf_ref[...]
    s_ref[...] = _pack2(
        jnp.dot(nf, wst_ref[...], preferred_element_type=jnp.float32))
    d_ref[...] = _pack2(
        jnp.dot(nf, wdt_ref[...], preferred_element_type=jnp.float32))


def _proj_call(nfeat, wst, wdt):
    return pl.pallas_call(
        _proj_body,
        out_shape=(
            jax.ShapeDtypeStruct((_N, _HW), jnp.int32),
            jax.ShapeDtypeStruct((_N, _HW), jnp.int32),
        ),
    )(nfeat, wst, wdt)


def _gather_call(mlp_s, mlp_d, src, dst, e_slice, c, n_chunks, c_tail=None):
    ct = c if c_tail is None else c_tail
    per_w = e_slice // _NW
    assert c % 8 == 0 and 0 < c <= 128 and per_w == c * (n_chunks - 1) + ct
    assert ct % 16 == 0 and ct <= c
    assert n_chunks % 2 == 1 and n_chunks >= 5

    def body(s_hbm, d_hbm, src_hbm, dst_hbm, out_hbm,
             idx_s, idx_d, buf_s, buf_d, obuf, sem_s, sem_d, sem_o):
        wid = lax.axis_index("s") * _NC + lax.axis_index("c")
        base = wid * per_w

        pltpu.sync_copy(src_hbm.at[pl.ds(pl.multiple_of(base, 8), per_w)],
                        idx_s)
        pltpu.sync_copy(dst_hbm.at[pl.ds(pl.multiple_of(base, 8), per_w)],
                        idx_d)

        def issue(j, slot, cj=c):
            js = pl.multiple_of(j * c, 8)
            pltpu.async_copy(s_hbm.at[idx_s.at[pl.ds(js, cj)]],
                             buf_s.at[slot, pl.ds(0, cj)], sem_s.at[slot])
            pltpu.async_copy(d_hbm.at[idx_d.at[pl.ds(js, cj)]],
                             buf_d.at[slot, pl.ds(0, cj)], sem_d.at[slot])

        def process(j, slot, first, cj=c):
            off = pl.multiple_of(base + j * c, 8)
            pltpu.make_async_copy(s_hbm.at[idx_s.at[pl.ds(0, cj)]],
                                  buf_s.at[slot, pl.ds(0, cj)],
                                  sem_s.at[slot]).wait()
            pltpu.make_async_copy(d_hbm.at[idx_d.at[pl.ds(0, cj)]],
                                  buf_d.at[slot, pl.ds(0, cj)],
                                  sem_d.at[slot]).wait()
            if not first:
                pltpu.make_async_copy(obuf.at[slot],
                                      out_hbm.at[pl.ds(0, c // 2)],
                                      sem_o.at[slot]).wait()

            def row(p, c2):
                for q in range(2):
                    e = p * 2 + q
                    for k in range(_HW // 16):
                        a = plsc.bitcast(buf_s[slot, e, pl.ds(k * 16, 16)],
                                         jnp.bfloat16)
                        b = plsc.bitcast(buf_d[slot, e, pl.ds(k * 16, 16)],
                                         jnp.bfloat16)
                        obuf[slot, p, pl.ds(q * _HW + k * 16, 16)] = (
                            plsc.bitcast(a + b, jnp.int32))
                return c2

            lax.fori_loop(0, cj // 2, row, 0)
            pltpu.async_copy(obuf.at[slot, pl.ds(0, cj // 2)],
                             out_hbm.at[pl.ds(off // 2, cj // 2)],
                             sem_o.at[slot])

        issue(0, 0)
        issue(1, 1)
        process(0, 0, first=True)
        issue(2, 0)
        process(1, 1, first=True)
        issue(3, 1)

        def pair(k, carry):
            j = k * 2
            process(j, 0, first=False)
            issue(j + 2, 0)
            process(j + 1, 1, first=False)
            issue(j + 3, 1)
            return carry

        lax.fori_loop(1, (n_chunks - 3) // 2, pair, 0)
        process(n_chunks - 3, 0, first=False)
        issue(n_chunks - 1, 0, ct)
        process(n_chunks - 2, 1, first=False)
        process(n_chunks - 1, 0, first=False, cj=ct)
        pltpu.make_async_copy(obuf.at[0, pl.ds(0, ct // 2)],
                              out_hbm.at[pl.ds(0, ct // 2)],
                              sem_o.at[0]).wait()
        pltpu.make_async_copy(obuf.at[1], out_hbm.at[pl.ds(0, c // 2)],
                              sem_o.at[1]).wait()

    mesh = plsc.VectorSubcoreMesh(
        core_axis_name="c", subcore_axis_name="s",
        num_cores=_NC, num_subcores=_NS)
    fn = pl.kernel(
        body,
        out_type=jax.ShapeDtypeStruct((e_slice // 2, _H), jnp.int32),
        mesh=mesh,
        compiler_params=pltpu.CompilerParams(
            use_tc_tiling_on_sc=False, needs_layout_passes=False),
        scratch_types=[
            pltpu.VMEM((per_w,), jnp.int32),
            pltpu.VMEM((per_w,), jnp.int32),
            pltpu.VMEM((2, c, _HW), jnp.int32),
            pltpu.VMEM((2, c, _HW), jnp.int32),
            pltpu.VMEM((2, c // 2, _H), jnp.int32),
            pltpu.SemaphoreType.DMA((2,)),
            pltpu.SemaphoreType.DMA((2,)),
            pltpu.SemaphoreType.DMA((2,)),
        ],
    )
    return fn(mlp_s, mlp_d, src, dst)


def _edge_math(e, gg, wet, wot, b1, bo, lng, lnb):
    even = _unpack2(gg[:, :_HW])
    odd = _unpack2(gg[:, _HW:])
    gnat = jnp.concatenate([even, odd], axis=-1).reshape(2 * gg.shape[0], _D)
    h = jnp.dot(e, wet, preferred_element_type=jnp.float32)
    h = h + gnat + b1
    h = h / (1.0 + jnp.exp(-h))
    o = jnp.dot(h, wot, preferred_element_type=jnp.float32)
    o = o + bo
    m = jnp.mean(o, axis=-1, keepdims=True)
    v = jnp.mean(o * o, axis=-1, keepdims=True) - m * m
    o = (o - m) * lax.rsqrt(v + _LN_EPS) * lng + lnb
    return o + e


def _edge_body_first(e_ref, g_ref, wet_ref, wot_ref, b1_ref, bo_ref,
                     lng_ref, lnb_ref, out_ref):
    out_ref[...] = _edge_math(
        e_ref[...], g_ref[...], wet_ref[...], wot_ref[...], b1_ref[...],
        bo_ref[...], lng_ref[...], lnb_ref[...])


def _edge_body_rest(prev_ref, e_ref, g_ref, wet_ref, wot_ref, b1_ref, bo_ref,
                    lng_ref, lnb_ref, out_ref):
    del prev_ref
    out_ref[...] = _edge_math(
        e_ref[...], g_ref[...], wet_ref[...], wot_ref[...], b1_ref[...],
        bo_ref[...], lng_ref[...], lnb_ref[...])


def _edge_call(efeat, g, wk, prev_out, e_off, be, n_blks):
    blk_off = e_off // be
    blk = pl.BlockSpec((be, _D), lambda i: (i + blk_off, 0))
    gblk = pl.BlockSpec((be // 2, _H), lambda i: (i, 0))
    full = pl.BlockSpec((_D, _H), lambda i: (0, 0))
    vec = pl.BlockSpec((1, _H), lambda i: (0, 0))
    out_shape = jax.ShapeDtypeStruct((_E, _D), jnp.float32)
    if prev_out is None:
        return pl.pallas_call(
            _edge_body_first,
            grid=(n_blks,),
            in_specs=[blk, gblk, full, full, vec, vec, vec, vec],
            out_specs=blk,
            out_shape=out_shape,
        )(efeat, g, *wk)
    return pl.pallas_call(
        _edge_body_rest,
        grid=(n_blks,),
        in_specs=[pl.BlockSpec(memory_space=pl.ANY),
                  blk, gblk, full, full, vec, vec, vec, vec],
        out_specs=blk,
        out_shape=out_shape,
        input_output_aliases={0: 0},
    )(prev_out, efeat, g, *wk)


def kernel(efeat, nfeat, src, dst, W_e, W_s, W_d, b1, W_o, b_o, ln_g, ln_b):
    mlp_s, mlp_d = _proj_call(nfeat, W_s.T, W_d.T)
    wk = (W_e.T, W_o.T, b1.reshape(1, _H), b_o.reshape(1, _D),
          ln_g.reshape(1, _D), ln_b.reshape(1, _D))
    gs = []
    e0 = 0
    for es, c, nch, _, ct in _SLICES:
        gs.append(_gather_call(mlp_s, mlp_d, src[e0:e0 + es],
                               dst[e0:e0 + es], es, c, nch, ct))
        e0 += es
    out = None
    e0 = 0
    for k, (es, _, _, be, _) in enumerate(_SLICES):
        out = _edge_call(efeat, gs[k], wk, out, e0, be, es // be)
        e0 += es
    return (out, nfeat)
